# speedup vs baseline: 2.0937x; 2.0937x over previous
import functools

import jax
import jax.numpy as jnp
from jax import lax
from jax.experimental import pallas as pl
from jax.experimental.pallas import tpu as pltpu
from jax.experimental.pallas import tpu_sc as plsc

B = 4096
L = 200
D = 128
NW = 32
CW = 128
ROWS_PER_W = (B * L) // NW
CHUNKS = ROWS_PER_W // CW
LANES = 16


@jax.jit
def _sc_embed(x3, table, pos):
  mesh = plsc.VectorSubcoreMesh(core_axis_name="c", subcore_axis_name="s")

  @functools.partial(
      pl.kernel,
      out_type=jax.ShapeDtypeStruct((B * L, D), jnp.float32),
      mesh=mesh,
      scratch_types=[
          pltpu.VMEM((CHUNKS, CW), jnp.int32),
          pltpu.VMEM((L, D), jnp.float32),
          pltpu.VMEM((CW, D), jnp.float32),
      ],
  )
  def k(x_hbm, tab_hbm, pos_hbm, out_hbm, idx_v, pos_v, buf_v):
    cid = lax.axis_index("c")
    sid = lax.axis_index("s")
    wid = sid * 2 + cid
    pltpu.sync_copy(x_hbm.at[wid], idx_v)
    pltpu.sync_copy(pos_hbm, pos_v)
    base = wid * ROWS_PER_W

    @pl.loop(0, CHUNKS)
    def _(c):
      fbase = base + c * CW
      pltpu.sync_copy(tab_hbm.at[idx_v.at[c]], buf_v)

      @pl.loop(0, CW)
      def _(j):
        lrow = lax.rem(fbase + j, L)
        for kk in range(0, D, LANES):
          slc = pl.ds(kk, LANES)
          buf_v[j, slc] = buf_v[j, slc] + pos_v[lrow, slc]

      pltpu.sync_copy(buf_v, out_hbm.at[pl.ds(fbase, CW)])

  return k(x3, table, pos)


def kernel(x, input_table, pos_table):
  x3 = x.astype(jnp.int32).reshape(NW, CHUNKS, CW)
  out = _sc_embed(x3, input_table, pos_table)
  return out.reshape(B, L, D)


# speedup vs baseline: 2.4845x; 1.1867x over previous
import functools

import jax
import jax.numpy as jnp
from jax import lax
from jax.experimental import pallas as pl
from jax.experimental.pallas import tpu as pltpu
from jax.experimental.pallas import tpu_sc as plsc

B = 4096
L = 200
D = 128
NW = 32
CW = 128
ROWS_PER_W = (B * L) // NW
CHUNKS = ROWS_PER_W // CW
LANES = 16


@jax.jit
def _sc_embed(x3, table, pos):
  mesh = plsc.VectorSubcoreMesh(core_axis_name="c", subcore_axis_name="s")

  @functools.partial(
      pl.kernel,
      out_type=jax.ShapeDtypeStruct((B * L, D), jnp.float32),
      mesh=mesh,
      scratch_types=[
          pltpu.VMEM((CHUNKS, CW), jnp.int32),
          pltpu.VMEM((L, D), jnp.float32),
          pltpu.VMEM((2, CW, D), jnp.float32),
          pltpu.SemaphoreType.DMA,
          pltpu.SemaphoreType.DMA,
          pltpu.SemaphoreType.DMA,
          pltpu.SemaphoreType.DMA,
      ],
  )
  def k(x_hbm, tab_hbm, pos_hbm, out_hbm, idx_v, pos_v, buf_v,
        gsem0, gsem1, wsem0, wsem1):
    cid = lax.axis_index("c")
    sid = lax.axis_index("s")
    wid = sid * 2 + cid
    pltpu.sync_copy(x_hbm.at[wid], idx_v)
    pltpu.sync_copy(pos_hbm, pos_v)
    base = wid * ROWS_PER_W

    def add_pos(slot, c):
      fbase = base + c * CW
      l0 = lax.rem(fbase, L)
      seg = jnp.minimum(L - l0, CW)

      def body(off):
        def f(j):
          lrow = j + off
          for kk in range(0, D, LANES):
            slc = pl.ds(kk, LANES)
            buf_v[slot, j, slc] = buf_v[slot, j, slc] + pos_v[lrow, slc]
        return f

      pl.loop(0, seg)(body(l0))
      pl.loop(seg, CW)(body(l0 - L))

    def start_gather(slot, c, sem):
      pltpu.async_copy(tab_hbm.at[idx_v.at[c]], buf_v.at[slot], sem)

    def start_write(slot, c, sem):
      pltpu.async_copy(buf_v.at[slot], out_hbm.at[pl.ds(base + c * CW, CW)],
                       sem)

    def wait_gather(slot, c, sem):
      pltpu.make_async_copy(tab_hbm.at[idx_v.at[c]], buf_v.at[slot],
                            sem).wait()

    def wait_write(slot, c, sem):
      pltpu.make_async_copy(buf_v.at[slot],
                            out_hbm.at[pl.ds(base + c * CW, CW)], sem).wait()

    start_gather(0, 0, gsem0)

    @pl.loop(0, CHUNKS, step=2)
    def _(c):
      wait_gather(0, c, gsem0)

      @pl.when(c > 0)
      def _():
        wait_write(1, c - 1, wsem1)

      start_gather(1, c + 1, gsem1)
      add_pos(0, c)
      start_write(0, c, wsem0)
      wait_gather(1, c + 1, gsem1)

      @pl.when(c + 2 < CHUNKS)
      def _():
        wait_write(0, c, wsem0)
        start_gather(0, c + 2, gsem0)

      add_pos(1, c + 1)
      start_write(1, c + 1, wsem1)

    wait_write(0, CHUNKS - 2, wsem0)
    wait_write(1, CHUNKS - 1, wsem1)

  return k(x3, table, pos)


def kernel(x, input_table, pos_table):
  x3 = x.astype(jnp.int32).reshape(NW, CHUNKS, CW)
  out = _sc_embed(x3, input_table, pos_table)
  return out.reshape(B, L, D)


# speedup vs baseline: 3.1293x; 1.2595x over previous
import functools

import jax
import jax.numpy as jnp
from jax import lax
from jax.experimental import pallas as pl
from jax.experimental.pallas import tpu as pltpu
from jax.experimental.pallas import tpu_sc as plsc

B = 4096
L = 200
D = 128
NW = 32
CW = 128
ROWS_PER_W = (B * L) // NW
CHUNKS = ROWS_PER_W // CW
LANES = 16


@jax.jit
def _sc_embed(x3, table, pos):
  mesh = plsc.VectorSubcoreMesh(core_axis_name="c", subcore_axis_name="s")

  @functools.partial(
      pl.kernel,
      out_type=jax.ShapeDtypeStruct((B * L, D), jnp.float32),
      mesh=mesh,
      scratch_types=[
          pltpu.VMEM((CHUNKS, CW), jnp.int32),
          pltpu.VMEM((L, D), jnp.float32),
          pltpu.VMEM((2, CW, D), jnp.float32),
          pltpu.SemaphoreType.DMA,
          pltpu.SemaphoreType.DMA,
          pltpu.SemaphoreType.DMA,
          pltpu.SemaphoreType.DMA,
      ],
  )
  def k(x_hbm, tab_hbm, pos_hbm, out_hbm, idx_v, pos_v, buf_v,
        gsem0, gsem1, wsem0, wsem1):
    cid = lax.axis_index("c")
    sid = lax.axis_index("s")
    wid = sid * 2 + cid
    pltpu.sync_copy(x_hbm.at[wid], idx_v)
    pltpu.sync_copy(pos_hbm, pos_v)
    base = wid * ROWS_PER_W

    def add_pos(slot, c):
      fbase = base + c * CW
      l0 = lax.rem(fbase, L)
      seg = jnp.minimum(L - l0, CW)

      def body(off):
        def f(j):
          lrow = j + off
          for kk in range(0, D, LANES):
            slc = pl.ds(kk, LANES)
            plsc.addupdate(buf_v.at[slot, j, slc], pos_v[lrow, slc])
        return f

      pl.loop(0, seg)(body(l0))
      pl.loop(seg, CW)(body(l0 - L))

    def start_gather(slot, c, sem):
      pltpu.async_copy(tab_hbm.at[idx_v.at[c]], buf_v.at[slot], sem)

    def start_write(slot, c, sem):
      pltpu.async_copy(buf_v.at[slot], out_hbm.at[pl.ds(base + c * CW, CW)],
                       sem)

    def wait_gather(slot, c, sem):
      pltpu.make_async_copy(tab_hbm.at[idx_v.at[c]], buf_v.at[slot],
                            sem).wait()

    def wait_write(slot, c, sem):
      pltpu.make_async_copy(buf_v.at[slot],
                            out_hbm.at[pl.ds(base + c * CW, CW)], sem).wait()

    start_gather(0, 0, gsem0)

    @pl.loop(0, CHUNKS, step=2)
    def _(c):
      wait_gather(0, c, gsem0)

      @pl.when(c > 0)
      def _():
        wait_write(1, c - 1, wsem1)

      start_gather(1, c + 1, gsem1)
      add_pos(0, c)
      start_write(0, c, wsem0)
      wait_gather(1, c + 1, gsem1)

      @pl.when(c + 2 < CHUNKS)
      def _():
        wait_write(0, c, wsem0)
        start_gather(0, c + 2, gsem0)

      add_pos(1, c + 1)
      start_write(1, c + 1, wsem1)

    wait_write(0, CHUNKS - 2, wsem0)
    wait_write(1, CHUNKS - 1, wsem1)

  return k(x3, table, pos)


def kernel(x, input_table, pos_table):
  x3 = x.astype(jnp.int32).reshape(NW, CHUNKS, CW)
  out = _sc_embed(x3, input_table, pos_table)
  return out.reshape(B, L, D)


# speedup vs baseline: 7.3403x; 2.3456x over previous
import functools

import jax
import jax.numpy as jnp
from jax import lax
from jax.experimental import pallas as pl
from jax.experimental.pallas import tpu as pltpu
from jax.experimental.pallas import tpu_sc as plsc

B = 4096
L = 200
D = 128
NW = 32
CW = 128
ROWS_PER_W = (B * L) // NW
CHUNKS = ROWS_PER_W // CW
LANES = 16


@jax.jit
def _sc_embed(x3, table, pos):
  mesh = plsc.VectorSubcoreMesh(core_axis_name="c", subcore_axis_name="s")

  @functools.partial(
      pl.kernel,
      out_type=jax.ShapeDtypeStruct((B * L, D), jnp.float32),
      mesh=mesh,
      scratch_types=[
          pltpu.VMEM((CHUNKS, CW), jnp.int32),
          pltpu.VMEM((2 * L, D), jnp.float32),
          pltpu.VMEM((2, CW, D), jnp.float32),
          pltpu.SemaphoreType.DMA,
          pltpu.SemaphoreType.DMA,
          pltpu.SemaphoreType.DMA,
          pltpu.SemaphoreType.DMA,
      ],
  )
  def k(x_hbm, tab_hbm, pos_hbm, out_hbm, idx_v, pos_v, buf_v,
        gsem0, gsem1, wsem0, wsem1):
    cid = lax.axis_index("c")
    sid = lax.axis_index("s")
    wid = sid * 2 + cid
    pltpu.sync_copy(x_hbm.at[wid], idx_v)
    pltpu.sync_copy(pos_hbm, pos_v)
    base = wid * ROWS_PER_W

    def add_pos(slot, c):
      l0 = lax.rem(base + c * CW, L)

      @plsc.parallel_loop(0, CW, unroll=4)
      def _(j):
        lrow = l0 + j
        for kk in range(0, D, LANES):
          slc = pl.ds(kk, LANES)
          plsc.addupdate(buf_v.at[slot, j, slc], pos_v[lrow, slc])

    def start_gather(slot, c, sem):
      pltpu.async_copy(tab_hbm.at[idx_v.at[c]], buf_v.at[slot], sem)

    def start_write(slot, c, sem):
      pltpu.async_copy(buf_v.at[slot], out_hbm.at[pl.ds(base + c * CW, CW)],
                       sem)

    def wait_gather(slot, c, sem):
      pltpu.make_async_copy(tab_hbm.at[idx_v.at[c]], buf_v.at[slot],
                            sem).wait()

    def wait_write(slot, c, sem):
      pltpu.make_async_copy(buf_v.at[slot],
                            out_hbm.at[pl.ds(base + c * CW, CW)], sem).wait()

    start_gather(0, 0, gsem0)

    @pl.loop(0, CHUNKS, step=2)
    def _(c):
      wait_gather(0, c, gsem0)

      @pl.when(c > 0)
      def _():
        wait_write(1, c - 1, wsem1)

      start_gather(1, c + 1, gsem1)
      add_pos(0, c)
      start_write(0, c, wsem0)
      wait_gather(1, c + 1, gsem1)

      @pl.when(c + 2 < CHUNKS)
      def _():
        wait_write(0, c, wsem0)
        start_gather(0, c + 2, gsem0)

      add_pos(1, c + 1)
      start_write(1, c + 1, wsem1)

    wait_write(0, CHUNKS - 2, wsem0)
    wait_write(1, CHUNKS - 1, wsem1)

  return k(x3, table, pos)


def kernel(x, input_table, pos_table):
  x3 = x.astype(jnp.int32).reshape(NW, CHUNKS, CW)
  pos2 = jnp.concatenate([pos_table, pos_table], axis=0)
  out = _sc_embed(x3, input_table, pos2)
  return out.reshape(B, L, D)
